# baseline (speedup 1.0000x reference)
import functools
import jax
import jax.numpy as jnp
from jax import lax
from jax.experimental import pallas as pl
from jax.experimental.pallas import tpu as pltpu
from jax.experimental.pallas import tpu_sc as plsc

B = 16384
D = 64
N = 1000000
NW = 32
BPW = B // NW
BLK = 20480
GRID = (N + BLK - 1) // BLK


def _tc_body(wu_ref, wm_ref, ut_ref, mt_ref, su_hbm, sm_hbm, stage, sems):
    i = pl.program_id(0)
    p = lax.rem(i, 2)

    @pl.when(i >= 2)
    def _drain_prev():
        pltpu.make_async_copy(stage.at[p, 0], su_hbm.at[pl.ds(0, BLK)], sems.at[p]).wait()
        pltpu.make_async_copy(stage.at[p, 1], sm_hbm.at[pl.ds(0, BLK)], sems.at[p]).wait()

    stage[p, 0, :] = jnp.dot(wu_ref[...], ut_ref[...],
                             preferred_element_type=jnp.float32)[0]
    stage[p, 1, :] = jnp.dot(wm_ref[...], mt_ref[...],
                             preferred_element_type=jnp.float32)[0]
    base = pl.multiple_of(i * BLK, BLK)
    pltpu.async_copy(stage.at[p, 0], su_hbm.at[pl.ds(base, BLK)], sems.at[p])
    pltpu.async_copy(stage.at[p, 1], sm_hbm.at[pl.ds(base, BLK)], sems.at[p])

    @pl.when(i == GRID - 1)
    def _drain_tail():
        pltpu.make_async_copy(stage.at[p, 0], su_hbm.at[pl.ds(0, BLK)], sems.at[p]).wait()
        pltpu.make_async_copy(stage.at[p, 1], sm_hbm.at[pl.ds(0, BLK)], sems.at[p]).wait()
        q = 1 - p
        pltpu.make_async_copy(stage.at[q, 0], su_hbm.at[pl.ds(0, BLK)], sems.at[q]).wait()
        pltpu.make_async_copy(stage.at[q, 1], sm_hbm.at[pl.ds(0, BLK)], sems.at[q]).wait()


def _tc_scan(ut_t, mt_t, wu, wm):
    return pl.pallas_call(
        _tc_body,
        grid=(GRID,),
        in_specs=[
            pl.BlockSpec((1, D), lambda i: (0, 0)),
            pl.BlockSpec((1, D), lambda i: (0, 0)),
            pl.BlockSpec((D, BLK), lambda i: (0, i)),
            pl.BlockSpec((D, BLK), lambda i: (0, i)),
        ],
        out_specs=[
            pl.BlockSpec(memory_space=pl.ANY),
            pl.BlockSpec(memory_space=pl.ANY),
        ],
        out_shape=[
            jax.ShapeDtypeStruct((GRID * BLK,), jnp.float32),
            jax.ShapeDtypeStruct((GRID * BLK,), jnp.float32),
        ],
        scratch_shapes=[
            pltpu.VMEM((2, 2, BLK), jnp.float32),
            pltpu.SemaphoreType.DMA((2,)),
        ],
    )(wu, wm, ut_t, mt_t)


def _sc_body(uid_hbm, mid_hbm, su_hbm, sm_hbm, bias_hbm, out_hbm,
             idx_u, idx_m, g_u, g_m, bv, outv, sem):
    wid = lax.axis_index("s") * 2 + lax.axis_index("c")
    base = wid * BPW

    pltpu.sync_copy(uid_hbm.at[pl.ds(base, BPW)], idx_u)
    pltpu.sync_copy(mid_hbm.at[pl.ds(base, BPW)], idx_m)
    pltpu.sync_copy(bias_hbm, bv)

    copies = []
    for j in range(BPW // 128):
        sl = pl.ds(j * 128, 128)
        copies.append(pltpu.async_copy(su_hbm.at[idx_u.at[sl]], g_u.at[sl], sem))
        copies.append(pltpu.async_copy(sm_hbm.at[idx_m.at[sl]], g_m.at[sl], sem))
    for c in copies:
        c.wait()

    bvec = bv[pl.ds(0, 16)]

    def body(g, carry):
        sl = pl.ds(g * 16, 16)
        outv[sl] = g_u[sl] + g_m[sl] + bvec
        return carry

    lax.fori_loop(0, BPW // 16, body, 0)
    pltpu.sync_copy(outv, out_hbm.at[pl.ds(base, BPW)])


def _sc_gather(uid, mid, su, sm, bias16):
    mesh = plsc.VectorSubcoreMesh(core_axis_name="c", subcore_axis_name="s")
    return pl.kernel(
        _sc_body,
        out_type=jax.ShapeDtypeStruct((B,), jnp.float32),
        mesh=mesh,
        scratch_types=[
            pltpu.VMEM((BPW,), jnp.int32),
            pltpu.VMEM((BPW,), jnp.int32),
            pltpu.VMEM((BPW,), jnp.float32),
            pltpu.VMEM((BPW,), jnp.float32),
            pltpu.VMEM((16,), jnp.float32),
            pltpu.VMEM((BPW,), jnp.float32),
            pltpu.SemaphoreType.DMA,
        ],
    )(uid, mid, su, sm, bias16)


@jax.jit
def kernel(user_id, movie_id, user_table, movie_table, W, b):
    uid = user_id.astype(jnp.int32)
    mid = movie_id.astype(jnp.int32)
    wu = W[:D, 0][None, :]
    wm = W[D:, 0][None, :]
    su, sm = _tc_scan(user_table.T, movie_table.T, wu, wm)
    bias16 = jnp.full((16,), b[0], jnp.float32)
    out = _sc_gather(uid, mid, su, sm, bias16)
    return out[:, None]


# speedup vs baseline: 1.0045x; 1.0045x over previous
import jax
import jax.numpy as jnp
from jax import lax
from jax.experimental import pallas as pl
from jax.experimental.pallas import tpu as pltpu
from jax.experimental.pallas import tpu_sc as plsc

B = 16384
D = 64
N = 1000000
NW = 32
BPW = B // NW
BLK = 20480
GRID = (N + BLK - 1) // BLK


def _tc_body(wu_ref, wm_ref, ut_ref, mt_ref, su_ref, sm_ref):
    su_ref[...] = jnp.dot(wu_ref[...], ut_ref[...],
                          preferred_element_type=jnp.float32)[0]
    sm_ref[...] = jnp.dot(wm_ref[...], mt_ref[...],
                          preferred_element_type=jnp.float32)[0]


def _tc_scan(ut_t, mt_t, wu, wm):
    return pl.pallas_call(
        _tc_body,
        grid=(GRID,),
        in_specs=[
            pl.BlockSpec((1, D), lambda i: (0, 0)),
            pl.BlockSpec((1, D), lambda i: (0, 0)),
            pl.BlockSpec((D, BLK), lambda i: (0, i)),
            pl.BlockSpec((D, BLK), lambda i: (0, i)),
        ],
        out_specs=[
            pl.BlockSpec((BLK,), lambda i: (i,)),
            pl.BlockSpec((BLK,), lambda i: (i,)),
        ],
        out_shape=[
            jax.ShapeDtypeStruct((N,), jnp.float32),
            jax.ShapeDtypeStruct((N,), jnp.float32),
        ],
    )(wu, wm, ut_t, mt_t)


def _sc_body(uid_hbm, mid_hbm, su_hbm, sm_hbm, bias_hbm, out_hbm,
             idx_u, idx_m, g_u, g_m, bv, outv, sem):
    wid = lax.axis_index("s") * 2 + lax.axis_index("c")
    base = wid * BPW

    pltpu.sync_copy(uid_hbm.at[pl.ds(base, BPW)], idx_u)
    pltpu.sync_copy(mid_hbm.at[pl.ds(base, BPW)], idx_m)
    pltpu.sync_copy(bias_hbm, bv)

    copies = []
    for j in range(BPW // 128):
        sl = pl.ds(j * 128, 128)
        copies.append(pltpu.async_copy(su_hbm.at[idx_u.at[sl]], g_u.at[sl], sem))
        copies.append(pltpu.async_copy(sm_hbm.at[idx_m.at[sl]], g_m.at[sl], sem))
    for c in copies:
        c.wait()

    bvec = bv[pl.ds(0, 16)]

    def body(g, carry):
        sl = pl.ds(g * 16, 16)
        outv[sl] = g_u[sl] + g_m[sl] + bvec
        return carry

    lax.fori_loop(0, BPW // 16, body, 0)
    pltpu.sync_copy(outv, out_hbm.at[pl.ds(base, BPW)])


def _sc_gather(uid, mid, su, sm, bias16):
    mesh = plsc.VectorSubcoreMesh(core_axis_name="c", subcore_axis_name="s")
    return pl.kernel(
        _sc_body,
        out_type=jax.ShapeDtypeStruct((B,), jnp.float32),
        mesh=mesh,
        scratch_types=[
            pltpu.VMEM((BPW,), jnp.int32),
            pltpu.VMEM((BPW,), jnp.int32),
            pltpu.VMEM((BPW,), jnp.float32),
            pltpu.VMEM((BPW,), jnp.float32),
            pltpu.VMEM((16,), jnp.float32),
            pltpu.VMEM((BPW,), jnp.float32),
            pltpu.SemaphoreType.DMA,
        ],
    )(uid, mid, su, sm, bias16)


@jax.jit
def kernel(user_id, movie_id, user_table, movie_table, W, b):
    uid = user_id.astype(jnp.int32)
    mid = movie_id.astype(jnp.int32)
    wu = W[:D, 0][None, :]
    wm = W[D:, 0][None, :]
    su, sm = _tc_scan(user_table.T, movie_table.T, wu, wm)
    bias16 = jnp.full((16,), b[0], jnp.float32)
    out = _sc_gather(uid, mid, su, sm, bias16)
    return out[:, None]
